# baseline (speedup 1.0000x reference)
import functools

import jax
import jax.numpy as jnp
from jax import lax
from jax.experimental import pallas as pl
from jax.experimental.pallas import tpu as pltpu
from jax.experimental.pallas import tpu_sc as plsc

B, H, W = 64, 512, 512
HW = H * W
D = 2
P = 4096
ROWS = D * B
RP = 2 * P
NC, NS = 2, 16
NW = NC * NS
RPT = ROWS // NW
L = 16
CHUNK = 1024
FIRE = RP // CHUNK


def _sc_body(xf, bidx, didx, out,
             bi0, bi1, di0, di1, gi0, gi1, va0, va1,
             sem_idx, sem_g0, sem_g1, sem_out):
    cid = lax.axis_index("c")
    sid = lax.axis_index("s")
    wid = sid * NC + cid

    bi = (bi0, bi1)
    di = (di0, di1)
    gi = (gi0, gi1)
    va = (va0, va1)

    def start_idx(j):
        r = wid * RPT + j
        pltpu.async_copy(bidx.at[r], bi[j % 2], sem_idx)
        pltpu.async_copy(didx.at[r], di[j % 2], sem_idx)

    def wait_idx(j):
        pltpu.make_async_copy(bidx.at[0], bi[j % 2], sem_idx).wait()
        pltpu.make_async_copy(didx.at[0], di[j % 2], sem_idx).wait()

    def reorder_fire(j, off):
        bi_v, di_v, gi_v, va_v = bi[j % 2], di[j % 2], gi[j % 2], va[j % 2]
        sem = sem_g0 if j % 2 == 0 else sem_g1
        ipc = CHUNK // 32
        for k in range(FIRE):
            for i in range(k * ipc, (k + 1) * ipc):
                bpos = (i >> 3) * 256 + (i & 7) * L
                gi_v[pl.ds(bpos, L)] = bi_v[pl.ds(i * L, L)] + off
                gi_v[pl.ds(bpos + 128, L)] = di_v[pl.ds(i * L, L)] + off
            o = k * CHUNK
            pltpu.async_copy(
                xf.at[gi_v.at[pl.ds(o, CHUNK)]],
                va_v.at[pl.ds(o, CHUNK)], sem)

    def drain_gather(j):
        gi_v, va_v = gi[j % 2], va[j % 2]
        sem = sem_g0 if j % 2 == 0 else sem_g1
        for k in range(FIRE):
            o = k * CHUNK
            pltpu.make_async_copy(
                xf.at[gi_v.at[pl.ds(o, CHUNK)]],
                va_v.at[pl.ds(o, CHUNK)], sem).wait()

    def start_out(j):
        r = wid * RPT + j
        pltpu.async_copy(va[j % 2], out.at[pl.ds(r * RP, RP)], sem_out)

    def wait_out(j):
        r = wid * RPT + j
        pltpu.make_async_copy(va[j % 2], out.at[pl.ds(r * RP, RP)], sem_out).wait()

    def row_off(j):
        return lax.rem(wid * RPT + j, B) * HW

    start_idx(0)
    wait_idx(0)
    start_idx(1)
    reorder_fire(0, row_off(0))
    for j in range(1, RPT):
        wait_idx(j)
        if j + 1 < RPT:
            start_idx(j + 1)
        if j >= 2:
            wait_out(j - 2)
        reorder_fire(j, row_off(j))
        drain_gather(j - 1)
        start_out(j - 1)
    drain_gather(RPT - 1)
    wait_out(RPT - 2)
    start_out(RPT - 1)
    wait_out(RPT - 1)


_gather_rows = functools.partial(
    pl.kernel,
    out_type=jax.ShapeDtypeStruct((ROWS * RP,), jnp.float32),
    mesh=plsc.VectorSubcoreMesh(core_axis_name="c", subcore_axis_name="s"),
    compiler_params=pltpu.CompilerParams(needs_layout_passes=False),
    scratch_types=[
        pltpu.VMEM((P,), jnp.int32),
        pltpu.VMEM((P,), jnp.int32),
        pltpu.VMEM((P,), jnp.int32),
        pltpu.VMEM((P,), jnp.int32),
        pltpu.VMEM((RP,), jnp.int32),
        pltpu.VMEM((RP,), jnp.int32),
        pltpu.VMEM((RP,), jnp.float32),
        pltpu.VMEM((RP,), jnp.float32),
        pltpu.SemaphoreType.DMA,
        pltpu.SemaphoreType.DMA,
        pltpu.SemaphoreType.DMA,
        pltpu.SemaphoreType.DMA,
    ],
)(_sc_body)


@jax.jit
def kernel(X, birth_idx, death_idx):
    xf = X.reshape(-1)
    bidx = birth_idx.astype(jnp.int32).reshape(ROWS, P)
    didx = death_idx.astype(jnp.int32).reshape(ROWS, P)
    out = _gather_rows(xf, bidx, didx)
    out5 = out.reshape(D, B, P // 128, 2, 128)
    return jnp.transpose(out5, (0, 1, 2, 4, 3)).reshape(D, B, P, 2)


# speedup vs baseline: 1.0304x; 1.0304x over previous
import functools

import jax
import jax.numpy as jnp
from jax import lax
from jax.experimental import pallas as pl
from jax.experimental.pallas import tpu as pltpu
from jax.experimental.pallas import tpu_sc as plsc

B, H, W = 64, 512, 512
HW = H * W
D = 2
P = 4096
ROWS = D * B
RP = 2 * P
NC, NS = 2, 16
NW = NC * NS
RPT = ROWS // NW
L = 16
CHUNK = 1024
FIRE = RP // CHUNK


def _sc_body(xf, bidx, didx, out,
             bi0, bi1, di0, di1, gi0, gi1, va0, va1,
             sem_idx, sem_g0, sem_g1, sem_out):
    cid = lax.axis_index("c")
    sid = lax.axis_index("s")
    wid = sid * NC + cid

    bi = (bi0, bi1)
    di = (di0, di1)
    gi = (gi0, gi1)
    va = (va0, va1)

    def start_idx(j):
        r = wid * RPT + j
        pltpu.async_copy(bidx.at[r], bi[j % 2], sem_idx)
        pltpu.async_copy(didx.at[r], di[j % 2], sem_idx)

    def wait_idx(j):
        pltpu.make_async_copy(bidx.at[0], bi[j % 2], sem_idx).wait()
        pltpu.make_async_copy(didx.at[0], di[j % 2], sem_idx).wait()

    def reorder_fire(j, off):
        bi_v, di_v, gi_v, va_v = bi[j % 2], di[j % 2], gi[j % 2], va[j % 2]
        sem = sem_g0 if j % 2 == 0 else sem_g1
        ipc = CHUNK // 32
        for k in range(FIRE):
            def body(i, carry):
                bpos = lax.shift_right_logical(i, 3) * 256 + (i & 7) * L
                gi_v[pl.ds(bpos, L)] = bi_v[pl.ds(i * L, L)] + off
                gi_v[pl.ds(bpos + 128, L)] = di_v[pl.ds(i * L, L)] + off
                return carry
            lax.fori_loop(k * ipc, (k + 1) * ipc, body, 0)
            o = k * CHUNK
            pltpu.async_copy(
                xf.at[gi_v.at[pl.ds(o, CHUNK)]],
                va_v.at[pl.ds(o, CHUNK)], sem)

    def drain_gather(j):
        gi_v, va_v = gi[j % 2], va[j % 2]
        sem = sem_g0 if j % 2 == 0 else sem_g1
        for k in range(FIRE):
            o = k * CHUNK
            pltpu.make_async_copy(
                xf.at[gi_v.at[pl.ds(o, CHUNK)]],
                va_v.at[pl.ds(o, CHUNK)], sem).wait()

    def start_out(j):
        r = wid * RPT + j
        pltpu.async_copy(va[j % 2], out.at[pl.ds(r * RP, RP)], sem_out)

    def wait_out(j):
        r = wid * RPT + j
        pltpu.make_async_copy(va[j % 2], out.at[pl.ds(r * RP, RP)], sem_out).wait()

    def row_off(j):
        return lax.rem(wid * RPT + j, B) * HW

    start_idx(0)
    wait_idx(0)
    start_idx(1)
    reorder_fire(0, row_off(0))
    for j in range(1, RPT):
        wait_idx(j)
        if j + 1 < RPT:
            start_idx(j + 1)
        if j >= 2:
            wait_out(j - 2)
        reorder_fire(j, row_off(j))
        drain_gather(j - 1)
        start_out(j - 1)
    drain_gather(RPT - 1)
    wait_out(RPT - 2)
    start_out(RPT - 1)
    wait_out(RPT - 1)


_gather_rows = functools.partial(
    pl.kernel,
    out_type=jax.ShapeDtypeStruct((ROWS * RP,), jnp.float32),
    mesh=plsc.VectorSubcoreMesh(core_axis_name="c", subcore_axis_name="s"),
    compiler_params=pltpu.CompilerParams(needs_layout_passes=False),
    scratch_types=[
        pltpu.VMEM((P,), jnp.int32),
        pltpu.VMEM((P,), jnp.int32),
        pltpu.VMEM((P,), jnp.int32),
        pltpu.VMEM((P,), jnp.int32),
        pltpu.VMEM((RP,), jnp.int32),
        pltpu.VMEM((RP,), jnp.int32),
        pltpu.VMEM((RP,), jnp.float32),
        pltpu.VMEM((RP,), jnp.float32),
        pltpu.SemaphoreType.DMA,
        pltpu.SemaphoreType.DMA,
        pltpu.SemaphoreType.DMA,
        pltpu.SemaphoreType.DMA,
    ],
)(_sc_body)


@jax.jit
def kernel(X, birth_idx, death_idx):
    xf = X.reshape(-1)
    bidx = birth_idx.astype(jnp.int32).reshape(ROWS, P)
    didx = death_idx.astype(jnp.int32).reshape(ROWS, P)
    out = _gather_rows(xf, bidx, didx)
    out5 = out.reshape(D, B, P // 128, 2, 128)
    return jnp.transpose(out5, (0, 1, 2, 4, 3)).reshape(D, B, P, 2)


# speedup vs baseline: 1.0433x; 1.0126x over previous
import functools

import jax
import jax.numpy as jnp
from jax import lax
from jax.experimental import pallas as pl
from jax.experimental.pallas import tpu as pltpu
from jax.experimental.pallas import tpu_sc as plsc

B, H, W = 64, 512, 512
HW = H * W
D = 2
P = 4096
ROWS = D * B
RP = 2 * P
NC, NS = 2, 16
NW = NC * NS
RPT = ROWS // NW
L = 16
CHUNK = 2048
FIRE = RP // CHUNK


def _sc_body(xf, bidx, didx, out,
             bi0, bi1, di0, di1, gi0, gi1, va0, va1,
             sem_idx, sem_g0, sem_g1, sem_out):
    cid = lax.axis_index("c")
    sid = lax.axis_index("s")
    wid = sid * NC + cid

    bi = (bi0, bi1)
    di = (di0, di1)
    gi = (gi0, gi1)
    va = (va0, va1)

    def start_idx(j):
        r = wid * RPT + j
        pltpu.async_copy(bidx.at[r], bi[j % 2], sem_idx)
        pltpu.async_copy(didx.at[r], di[j % 2], sem_idx)

    def wait_idx(j):
        pltpu.make_async_copy(bidx.at[0], bi[j % 2], sem_idx).wait()
        pltpu.make_async_copy(didx.at[0], di[j % 2], sem_idx).wait()

    def reorder_fire(j, off):
        bi_v, di_v, gi_v, va_v = bi[j % 2], di[j % 2], gi[j % 2], va[j % 2]
        sem = sem_g0 if j % 2 == 0 else sem_g1
        ipc = CHUNK // 32
        for k in range(FIRE):
            def body(i, carry):
                bpos = lax.shift_right_logical(i, 3) * 256 + (i & 7) * L
                gi_v[pl.ds(bpos, L)] = bi_v[pl.ds(i * L, L)] + off
                gi_v[pl.ds(bpos + 128, L)] = di_v[pl.ds(i * L, L)] + off
                return carry
            lax.fori_loop(k * ipc, (k + 1) * ipc, body, 0)
            o = k * CHUNK
            pltpu.async_copy(
                xf.at[gi_v.at[pl.ds(o, CHUNK)]],
                va_v.at[pl.ds(o, CHUNK)], sem)

    def drain_gather(j):
        gi_v, va_v = gi[j % 2], va[j % 2]
        sem = sem_g0 if j % 2 == 0 else sem_g1
        for k in range(FIRE):
            o = k * CHUNK
            pltpu.make_async_copy(
                xf.at[gi_v.at[pl.ds(o, CHUNK)]],
                va_v.at[pl.ds(o, CHUNK)], sem).wait()

    def start_out(j):
        r = wid * RPT + j
        pltpu.async_copy(va[j % 2], out.at[pl.ds(r * RP, RP)], sem_out)

    def wait_out(j):
        r = wid * RPT + j
        pltpu.make_async_copy(va[j % 2], out.at[pl.ds(r * RP, RP)], sem_out).wait()

    def row_off(j):
        return lax.rem(wid * RPT + j, B) * HW

    start_idx(0)
    wait_idx(0)
    start_idx(1)
    reorder_fire(0, row_off(0))
    for j in range(1, RPT):
        wait_idx(j)
        if j + 1 < RPT:
            start_idx(j + 1)
        if j >= 2:
            wait_out(j - 2)
        reorder_fire(j, row_off(j))
        drain_gather(j - 1)
        start_out(j - 1)
    drain_gather(RPT - 1)
    wait_out(RPT - 2)
    start_out(RPT - 1)
    wait_out(RPT - 1)


_gather_rows = functools.partial(
    pl.kernel,
    out_type=jax.ShapeDtypeStruct((ROWS * RP,), jnp.float32),
    mesh=plsc.VectorSubcoreMesh(core_axis_name="c", subcore_axis_name="s"),
    compiler_params=pltpu.CompilerParams(needs_layout_passes=False),
    scratch_types=[
        pltpu.VMEM((P,), jnp.int32),
        pltpu.VMEM((P,), jnp.int32),
        pltpu.VMEM((P,), jnp.int32),
        pltpu.VMEM((P,), jnp.int32),
        pltpu.VMEM((RP,), jnp.int32),
        pltpu.VMEM((RP,), jnp.int32),
        pltpu.VMEM((RP,), jnp.float32),
        pltpu.VMEM((RP,), jnp.float32),
        pltpu.SemaphoreType.DMA,
        pltpu.SemaphoreType.DMA,
        pltpu.SemaphoreType.DMA,
        pltpu.SemaphoreType.DMA,
    ],
)(_sc_body)


@jax.jit
def kernel(X, birth_idx, death_idx):
    xf = X.reshape(-1)
    bidx = birth_idx.astype(jnp.int32).reshape(ROWS, P)
    didx = death_idx.astype(jnp.int32).reshape(ROWS, P)
    out = _gather_rows(xf, bidx, didx)
    out5 = out.reshape(D, B, P // 128, 2, 128)
    return jnp.transpose(out5, (0, 1, 2, 4, 3)).reshape(D, B, P, 2)
